# baseline (speedup 1.0000x reference)
import functools
import jax
import jax.numpy as jnp
from jax import lax
from jax.experimental import pallas as pl
from jax.experimental.pallas import tpu as pltpu
from jax.experimental.pallas import tpu_sc as plsc

D = 128
N_USER = 100000
N_MOVIE = 50000
E = 500000

NC, NS = 2, 16
ND_PAD = 50176
NPASS = 4
RNG_ROWS = 6272
SUB_ROWS = RNG_ROWS // NS
ZB_ROWS = 56
TRASH = RNG_ROWS

E_PAD = 524288
E_ROWS = E_PAD // 128
CHUNK_ROWS = E_ROWS // NS
SR_ROWS = 128
BLK_ROWS = 16
NBLK = SR_ROWS // BLK_ROWS
SEL_ROWS = SR_ROWS + 8

CNT_PAD = 50432
CNT_SUB = CNT_PAD // NS
PAD_DST = ND_PAD

BM = 512


def _agg_body(table_hbm, src_hbm, dst_hbm, out_hbm,
              acc_sh, src_v, dst_v, sel_v, rows0_v, rows1_v,
              isrc0_v, idst0_v, isrc1_v, idst1_v, zero_v, wb_v, sem):
    c = lax.axis_index("c")
    s = lax.axis_index("s")
    zvec = jnp.zeros((16,), jnp.float32)
    ipad = jnp.full((16,), TRASH << 16, jnp.int32)
    iota = lax.iota(jnp.int32, 16)
    rows_b = (rows0_v, rows1_v)
    isrc_b = (isrc0_v, isrc1_v)
    idst_b = (idst0_v, idst1_v)

    def unpack(j, b):
        for g in range(8):
            sl = pl.ds(g * 16, 16)
            packed = sel_v[j, sl]
            isrc_b[b][sl] = jnp.bitwise_and(packed, 0xFFFF)
            idst_b[b][sl] = lax.shift_right_logical(packed, 16)

    def start_gather(b):
        return pltpu.async_copy(table_hbm.at[isrc_b[b]], rows_b[b], sem)

    def wait_gather(b):
        pltpu.make_async_copy(table_hbm.at[isrc_b[b]], rows_b[b], sem).wait()

    def zfill(r, _):
        for g in range(8):
            zero_v[r, pl.ds(g * 16, 16)] = zvec
        return 0
    lax.fori_loop(0, zero_v.shape[0], zfill, 0)

    for p in range(NPASS):
        rng = p * NC + c
        lo = rng * RNG_ROWS
        for t in range(SUB_ROWS // ZB_ROWS):
            pltpu.sync_copy(
                zero_v, acc_sh.at[pl.ds(s * SUB_ROWS + t * ZB_ROWS, ZB_ROWS)])
        plsc.subcore_barrier()

        for u in range(CHUNK_ROWS // SR_ROWS):
            cursor = jnp.zeros((16,), jnp.int32)
            for b in range(NBLK):
                base = s * CHUNK_ROWS + u * SR_ROWS + b * BLK_ROWS
                pltpu.sync_copy(src_hbm.at[pl.ds(base, BLK_ROWS)], src_v)
                pltpu.sync_copy(dst_hbm.at[pl.ds(base, BLK_ROWS)], dst_v)

                def vec_body(i, cur):
                    r = i // 8
                    g = i % 8
                    sl = pl.ds(g * 16, 16)
                    d = dst_v[r, sl]
                    sr = src_v[r, sl]
                    ud = d - lo
                    m = plsc.bitcast(ud, jnp.uint32) < jnp.uint32(RNG_ROWS)
                    cum = plsc.cumsum(m.astype(jnp.int32))
                    pos = cur + cum - 1
                    row = jnp.right_shift(pos, 7)
                    col = jnp.bitwise_and(pos, 127)
                    packed = jnp.bitwise_or(
                        sr, lax.shift_left(ud, jnp.full((16,), 16,
                                                        jnp.int32)))
                    plsc.store_scatter(sel_v, [row, col], packed, mask=m)
                    return cur + plsc.all_reduce_population_count(m)
                cursor = lax.fori_loop(0, BLK_ROWS * 8, vec_body, cursor)

            n_sel = jnp.max(cursor)
            for k in range(8):
                pos = n_sel + k * 16 + iota
                row = jnp.right_shift(pos, 7)
                col = jnp.bitwise_and(pos, 127)
                plsc.store_scatter(sel_v, [row, col], ipad)

            nb = (n_sel + 127) // 128

            @pl.when(nb > 0)
            def _prime():
                unpack(0, 0)
                start_gather(0)

            def pair_body(jj, _):
                for b in range(2):
                    j = jj * 2 + b

                    @pl.when(j < nb)
                    def _step():
                        @pl.when(j + 1 < nb)
                        def _next():
                            unpack(j + 1, 1 - b)
                            start_gather(1 - b)
                        wait_gather(b)
                        pltpu.sync_copy(rows_b[b], acc_sh.at[idst_b[b]],
                                        add=True)
                return 0
            lax.fori_loop(0, (nb + 1) // 2, pair_body, 0)
        plsc.subcore_barrier()

        for t in range(SUB_ROWS // ZB_ROWS):
            off = s * SUB_ROWS + t * ZB_ROWS
            pltpu.sync_copy(acc_sh.at[pl.ds(off, ZB_ROWS)], wb_v)
            pltpu.sync_copy(wb_v, out_hbm.at[pl.ds(lo + off, ZB_ROWS)])
        plsc.subcore_barrier()


def _cnt_body(dst_hbm, out_hbm, sh_cnt, dst_v, ones_v, stage_v):
    c = lax.axis_index("c")
    s = lax.axis_index("s")
    wid = s * NC + c
    izero = jnp.zeros((16,), jnp.int32)
    for g in range(8):
        ones_v[pl.ds(g * 16, 16)] = jnp.ones((16,), jnp.int32)
    for k in range(CNT_SUB // 16):
        stage_v[pl.ds(k * 16, 16)] = izero
    pltpu.sync_copy(stage_v, sh_cnt.at[pl.ds(s * CNT_SUB, CNT_SUB)])
    plsc.subcore_barrier()

    rows_per_tile = E_ROWS // (NC * NS)
    pltpu.sync_copy(dst_hbm.at[pl.ds(wid * rows_per_tile, rows_per_tile)],
                    dst_v)

    def row_body(r, _):
        pltpu.sync_copy(ones_v, sh_cnt.at[dst_v.at[r]], add=True)
        return 0
    lax.fori_loop(0, rows_per_tile, row_body, 0)
    plsc.subcore_barrier()
    pltpu.sync_copy(sh_cnt.at[pl.ds(s * CNT_SUB, CNT_SUB)], stage_v)
    pltpu.sync_copy(stage_v,
                    out_hbm.at[pl.ds(c * CNT_PAD + s * CNT_SUB, CNT_SUB)])


_SC_MESH = plsc.VectorSubcoreMesh(core_axis_name="c", subcore_axis_name="s")


@jax.jit
def _agg(table, src2d, dst2d):
    return pl.kernel(
        _agg_body,
        out_type=jax.ShapeDtypeStruct((ND_PAD, D), jnp.float32),
        mesh=_SC_MESH,
        scratch_types=[
            pltpu.VMEM_SHARED((RNG_ROWS + 16, D), jnp.float32),
            pltpu.VMEM((BLK_ROWS, 128), jnp.int32),
            pltpu.VMEM((BLK_ROWS, 128), jnp.int32),
            pltpu.VMEM((SEL_ROWS, 128), jnp.int32),
            pltpu.VMEM((128, D), jnp.float32),
            pltpu.VMEM((128, D), jnp.float32),
            pltpu.VMEM((128,), jnp.int32),
            pltpu.VMEM((128,), jnp.int32),
            pltpu.VMEM((128,), jnp.int32),
            pltpu.VMEM((128,), jnp.int32),
            pltpu.VMEM((ZB_ROWS, D), jnp.float32),
            pltpu.VMEM((ZB_ROWS, D), jnp.float32),
            pltpu.SemaphoreType.DMA,
        ],
        compiler_params=pltpu.CompilerParams(needs_layout_passes=False),
    )(table, src2d, dst2d)


@jax.jit
def _count(dst2d):
    return pl.kernel(
        _cnt_body,
        out_type=jax.ShapeDtypeStruct((NC * CNT_PAD,), jnp.int32),
        mesh=_SC_MESH,
        scratch_types=[
            pltpu.VMEM_SHARED((CNT_PAD,), jnp.int32),
            pltpu.VMEM((E_ROWS // (NC * NS), 128), jnp.int32),
            pltpu.VMEM((128,), jnp.int32),
            pltpu.VMEM((CNT_SUB,), jnp.int32),
        ],
    )(dst2d)


def _tc_body(cnt_ref, agg_ref, x_ref, wl_ref, bl_ref, wr_ref, o_ref,
             *, relu, nblk_agg):
    i = pl.program_id(0)
    valid = (i < nblk_agg).astype(jnp.float32)
    cnt = (cnt_ref[0, :] + cnt_ref[1, :]).astype(jnp.float32)
    inv = valid / jnp.maximum(cnt, 1.0)
    agg = agg_ref[...] * inv[:, None]
    acc = jnp.dot(agg, wl_ref[...], preferred_element_type=jnp.float32)
    acc = acc + jnp.dot(x_ref[...], wr_ref[...],
                        preferred_element_type=jnp.float32)
    acc = acc + bl_ref[...]
    if relu:
        acc = jnp.maximum(acc, 0.0)
    o_ref[...] = acc


def _tc_call(cnt, agg, x, Wl, bl, Wr, relu):
    n = x.shape[0]
    nblk_agg = ND_PAD // BM
    grid = pl.cdiv(n, BM)
    clamp = lambda i: jnp.minimum(i, nblk_agg - 1)
    return pl.pallas_call(
        functools.partial(_tc_body, relu=relu, nblk_agg=nblk_agg),
        grid=(grid,),
        in_specs=[
            pl.BlockSpec((2, BM), lambda i: (0, clamp(i))),
            pl.BlockSpec((BM, D), lambda i: (clamp(i), 0)),
            pl.BlockSpec((BM, D), lambda i: (i, 0)),
            pl.BlockSpec((D, D), lambda i: (0, 0)),
            pl.BlockSpec((1, D), lambda i: (0, 0)),
            pl.BlockSpec((D, D), lambda i: (0, 0)),
        ],
        out_specs=pl.BlockSpec((BM, D), lambda i: (i, 0)),
        out_shape=jax.ShapeDtypeStruct((n, D), jnp.float32),
    )(cnt, agg, x, Wl, bl, Wr)


def _pad_edges(edge_index):
    src = jnp.concatenate(
        [edge_index[0], jnp.zeros((E_PAD - E,), jnp.int32)]).reshape(E_ROWS, 128)
    dst = jnp.concatenate(
        [edge_index[1], jnp.full((E_PAD - E,), PAD_DST, jnp.int32)]
    ).reshape(E_ROWS, 128)
    return src, dst


def kernel(x_user, x_movie, edge_index_rates, edge_index_rev_rates,
           W1rl, b1rl, W1rr, W1vl, b1vl, W1vr,
           W2rl, b2rl, W2rr, W2vl, b2vl, W2vr):
    src_r, dst_r = _pad_edges(edge_index_rates)
    src_v, dst_v = _pad_edges(edge_index_rev_rates)
    cnt_r = _count(dst_r).reshape(NC, CNT_PAD)
    cnt_v = _count(dst_v).reshape(NC, CNT_PAD)

    agg1m = _agg(x_user, src_r, dst_r)
    agg1u = _agg(x_movie, src_v, dst_v)
    movie1 = _tc_call(cnt_r, agg1m, x_movie, W1rl, b1rl.reshape(1, D), W1rr,
                      relu=True)
    user1 = _tc_call(cnt_v, agg1u, x_user, W1vl, b1vl.reshape(1, D), W1vr,
                     relu=True)

    agg2m = _agg(user1, src_r, dst_r)
    agg2u = _agg(movie1, src_v, dst_v)
    movie2 = _tc_call(cnt_r, agg2m, movie1, W2rl, b2rl.reshape(1, D), W2rr,
                      relu=False)
    user2 = _tc_call(cnt_v, agg2u, user1, W2vl, b2vl.reshape(1, D), W2vr,
                     relu=False)
    return (user2, movie2)


# speedup vs baseline: 1.0288x; 1.0288x over previous
import functools
import jax
import jax.numpy as jnp
from jax import lax
from jax.experimental import pallas as pl
from jax.experimental.pallas import tpu as pltpu
from jax.experimental.pallas import tpu_sc as plsc

D = 128
N_USER = 100000
N_MOVIE = 50000
E = 500000

NC, NS = 2, 16
ND_PAD = 50176
NPASS = 4
RNG_ROWS = 6272
SUB_ROWS = RNG_ROWS // NS
ZB_ROWS = 56
TRASH = RNG_ROWS

E_PAD = 524288
E_ROWS = E_PAD // 128
CHUNK_ROWS = E_ROWS // NS
SR_ROWS = 128
BLK_ROWS = 16
NBLK = SR_ROWS // BLK_ROWS
SEL_ROWS = SR_ROWS + 8

CNT_PAD = 50432
CNT_SUB = CNT_PAD // NS
PAD_DST = ND_PAD

BM = 512


def _agg_body(table_hbm, src_hbm, dst_hbm, out_hbm,
              acc_sh, src_v, dst_v, sel_v, rows0_v, rows1_v,
              isrc0_v, idst0_v, isrc1_v, idst1_v, wb0_v, wb1_v, sem):
    c = lax.axis_index("c")
    s = lax.axis_index("s")
    zvec = jnp.zeros((16,), jnp.float32)
    ipad = jnp.full((16,), TRASH << 16, jnp.int32)
    iota = lax.iota(jnp.int32, 16)
    rows_b = (rows0_v, rows1_v)
    isrc_b = (isrc0_v, isrc1_v)
    idst_b = (idst0_v, idst1_v)

    def unpack(j, b):
        for g in range(8):
            sl = pl.ds(g * 16, 16)
            packed = sel_v[j, sl]
            isrc_b[b][sl] = jnp.bitwise_and(packed, 0xFFFF)
            idst_b[b][sl] = lax.shift_right_logical(packed, 16)

    def start_gather(b):
        return pltpu.async_copy(table_hbm.at[isrc_b[b]], rows_b[b], sem)

    def wait_gather(b):
        pltpu.make_async_copy(table_hbm.at[isrc_b[b]], rows_b[b], sem).wait()

    wb_b = (wb0_v, wb1_v)

    def zfill(r, _):
        for g in range(8):
            wb0_v[r, pl.ds(g * 16, 16)] = zvec
        return 0
    lax.fori_loop(0, ZB_ROWS, zfill, 0)

    nzb = SUB_ROWS // ZB_ROWS
    for p in range(NPASS):
        rng = p * NC + c
        lo = rng * RNG_ROWS
        for t in range(nzb):
            pltpu.async_copy(
                wb0_v, acc_sh.at[pl.ds(s * SUB_ROWS + t * ZB_ROWS, ZB_ROWS)],
                sem)
        for t in range(nzb):
            pltpu.make_async_copy(
                wb0_v, acc_sh.at[pl.ds(s * SUB_ROWS, ZB_ROWS)], sem).wait()
        plsc.subcore_barrier()

        for u in range(CHUNK_ROWS // SR_ROWS):
            cursor = jnp.zeros((16,), jnp.int32)
            for b in range(NBLK):
                base = s * CHUNK_ROWS + u * SR_ROWS + b * BLK_ROWS
                pltpu.async_copy(src_hbm.at[pl.ds(base, BLK_ROWS)], src_v,
                                 sem)
                pltpu.async_copy(dst_hbm.at[pl.ds(base, BLK_ROWS)], dst_v,
                                 sem)
                pltpu.make_async_copy(src_hbm.at[pl.ds(base, BLK_ROWS)],
                                      src_v, sem).wait()
                pltpu.make_async_copy(dst_hbm.at[pl.ds(base, BLK_ROWS)],
                                      dst_v, sem).wait()

                def vec_body(i, cur):
                    r = i // 8
                    g = i % 8
                    sl = pl.ds(g * 16, 16)
                    d = dst_v[r, sl]
                    sr = src_v[r, sl]
                    ud = d - lo
                    m = plsc.bitcast(ud, jnp.uint32) < jnp.uint32(RNG_ROWS)
                    cum = plsc.cumsum(m.astype(jnp.int32))
                    pos = cur + cum - 1
                    row = jnp.right_shift(pos, 7)
                    col = jnp.bitwise_and(pos, 127)
                    packed = jnp.bitwise_or(
                        sr, lax.shift_left(ud, jnp.full((16,), 16,
                                                        jnp.int32)))
                    plsc.store_scatter(sel_v, [row, col], packed, mask=m)
                    return cur + plsc.all_reduce_population_count(m)
                cursor = lax.fori_loop(0, BLK_ROWS * 8, vec_body, cursor)

            n_sel = jnp.max(cursor)
            for k in range(8):
                pos = n_sel + k * 16 + iota
                row = jnp.right_shift(pos, 7)
                col = jnp.bitwise_and(pos, 127)
                plsc.store_scatter(sel_v, [row, col], ipad)

            nb = (n_sel + 127) // 128

            @pl.when(nb > 0)
            def _prime():
                unpack(0, 0)
                start_gather(0)

            def pair_body(jj, _):
                for b in range(2):
                    j = jj * 2 + b

                    @pl.when(j < nb)
                    def _step():
                        @pl.when(j + 1 < nb)
                        def _next():
                            unpack(j + 1, 1 - b)
                            start_gather(1 - b)
                        wait_gather(b)
                        pltpu.sync_copy(rows_b[b], acc_sh.at[idst_b[b]],
                                        add=True)
                return 0
            lax.fori_loop(0, (nb + 1) // 2, pair_body, 0)
        plsc.subcore_barrier()

        for t in range(nzb):
            b = t % 2
            off = s * SUB_ROWS + t * ZB_ROWS
            if t >= 2:
                pltpu.make_async_copy(
                    wb_b[b], out_hbm.at[pl.ds(lo, ZB_ROWS)], sem).wait()
            pltpu.sync_copy(acc_sh.at[pl.ds(off, ZB_ROWS)], wb_b[b])
            pltpu.async_copy(wb_b[b], out_hbm.at[pl.ds(lo + off, ZB_ROWS)],
                             sem)
        for b in range(2):
            pltpu.make_async_copy(
                wb_b[b], out_hbm.at[pl.ds(lo, ZB_ROWS)], sem).wait()
        plsc.subcore_barrier()
        if p + 1 < NPASS:
            lax.fori_loop(0, ZB_ROWS, zfill, 0)


def _cnt_body(dst_hbm, out_hbm, sh_cnt, dst_v, ones_v, stage_v):
    c = lax.axis_index("c")
    s = lax.axis_index("s")
    wid = s * NC + c
    izero = jnp.zeros((16,), jnp.int32)
    for g in range(8):
        ones_v[pl.ds(g * 16, 16)] = jnp.ones((16,), jnp.int32)
    for k in range(CNT_SUB // 16):
        stage_v[pl.ds(k * 16, 16)] = izero
    pltpu.sync_copy(stage_v, sh_cnt.at[pl.ds(s * CNT_SUB, CNT_SUB)])
    plsc.subcore_barrier()

    rows_per_tile = E_ROWS // (NC * NS)
    pltpu.sync_copy(dst_hbm.at[pl.ds(wid * rows_per_tile, rows_per_tile)],
                    dst_v)

    def row_body(r, _):
        pltpu.sync_copy(ones_v, sh_cnt.at[dst_v.at[r]], add=True)
        return 0
    lax.fori_loop(0, rows_per_tile, row_body, 0)
    plsc.subcore_barrier()
    pltpu.sync_copy(sh_cnt.at[pl.ds(s * CNT_SUB, CNT_SUB)], stage_v)
    pltpu.sync_copy(stage_v,
                    out_hbm.at[pl.ds(c * CNT_PAD + s * CNT_SUB, CNT_SUB)])


_SC_MESH = plsc.VectorSubcoreMesh(core_axis_name="c", subcore_axis_name="s")


@jax.jit
def _agg(table, src2d, dst2d):
    return pl.kernel(
        _agg_body,
        out_type=jax.ShapeDtypeStruct((ND_PAD, D), jnp.float32),
        mesh=_SC_MESH,
        scratch_types=[
            pltpu.VMEM_SHARED((RNG_ROWS + 16, D), jnp.float32),
            pltpu.VMEM((BLK_ROWS, 128), jnp.int32),
            pltpu.VMEM((BLK_ROWS, 128), jnp.int32),
            pltpu.VMEM((SEL_ROWS, 128), jnp.int32),
            pltpu.VMEM((128, D), jnp.float32),
            pltpu.VMEM((128, D), jnp.float32),
            pltpu.VMEM((128,), jnp.int32),
            pltpu.VMEM((128,), jnp.int32),
            pltpu.VMEM((128,), jnp.int32),
            pltpu.VMEM((128,), jnp.int32),
            pltpu.VMEM((ZB_ROWS, D), jnp.float32),
            pltpu.VMEM((ZB_ROWS, D), jnp.float32),
            pltpu.SemaphoreType.DMA,
        ],
        compiler_params=pltpu.CompilerParams(needs_layout_passes=False),
    )(table, src2d, dst2d)


@jax.jit
def _count(dst2d):
    return pl.kernel(
        _cnt_body,
        out_type=jax.ShapeDtypeStruct((NC * CNT_PAD,), jnp.int32),
        mesh=_SC_MESH,
        scratch_types=[
            pltpu.VMEM_SHARED((CNT_PAD,), jnp.int32),
            pltpu.VMEM((E_ROWS // (NC * NS), 128), jnp.int32),
            pltpu.VMEM((128,), jnp.int32),
            pltpu.VMEM((CNT_SUB,), jnp.int32),
        ],
    )(dst2d)


def _tc_body(cnt_ref, agg_ref, x_ref, wl_ref, bl_ref, wr_ref, o_ref,
             *, relu, nblk_agg):
    i = pl.program_id(0)
    valid = (i < nblk_agg).astype(jnp.float32)
    cnt = (cnt_ref[0, :] + cnt_ref[1, :]).astype(jnp.float32)
    inv = valid / jnp.maximum(cnt, 1.0)
    agg = agg_ref[...] * inv[:, None]
    acc = jnp.dot(agg, wl_ref[...], preferred_element_type=jnp.float32)
    acc = acc + jnp.dot(x_ref[...], wr_ref[...],
                        preferred_element_type=jnp.float32)
    acc = acc + bl_ref[...]
    if relu:
        acc = jnp.maximum(acc, 0.0)
    o_ref[...] = acc


def _tc_call(cnt, agg, x, Wl, bl, Wr, relu):
    n = x.shape[0]
    nblk_agg = ND_PAD // BM
    grid = pl.cdiv(n, BM)
    clamp = lambda i: jnp.minimum(i, nblk_agg - 1)
    return pl.pallas_call(
        functools.partial(_tc_body, relu=relu, nblk_agg=nblk_agg),
        grid=(grid,),
        in_specs=[
            pl.BlockSpec((2, BM), lambda i: (0, clamp(i))),
            pl.BlockSpec((BM, D), lambda i: (clamp(i), 0)),
            pl.BlockSpec((BM, D), lambda i: (i, 0)),
            pl.BlockSpec((D, D), lambda i: (0, 0)),
            pl.BlockSpec((1, D), lambda i: (0, 0)),
            pl.BlockSpec((D, D), lambda i: (0, 0)),
        ],
        out_specs=pl.BlockSpec((BM, D), lambda i: (i, 0)),
        out_shape=jax.ShapeDtypeStruct((n, D), jnp.float32),
    )(cnt, agg, x, Wl, bl, Wr)


def _pad_edges(edge_index):
    src = jnp.concatenate(
        [edge_index[0], jnp.zeros((E_PAD - E,), jnp.int32)]).reshape(E_ROWS, 128)
    dst = jnp.concatenate(
        [edge_index[1], jnp.full((E_PAD - E,), PAD_DST, jnp.int32)]
    ).reshape(E_ROWS, 128)
    return src, dst


def kernel(x_user, x_movie, edge_index_rates, edge_index_rev_rates,
           W1rl, b1rl, W1rr, W1vl, b1vl, W1vr,
           W2rl, b2rl, W2rr, W2vl, b2vl, W2vr):
    src_r, dst_r = _pad_edges(edge_index_rates)
    src_v, dst_v = _pad_edges(edge_index_rev_rates)
    cnt_r = _count(dst_r).reshape(NC, CNT_PAD)
    cnt_v = _count(dst_v).reshape(NC, CNT_PAD)

    agg1m = _agg(x_user, src_r, dst_r)
    agg1u = _agg(x_movie, src_v, dst_v)
    movie1 = _tc_call(cnt_r, agg1m, x_movie, W1rl, b1rl.reshape(1, D), W1rr,
                      relu=True)
    user1 = _tc_call(cnt_v, agg1u, x_user, W1vl, b1vl.reshape(1, D), W1vr,
                     relu=True)

    agg2m = _agg(user1, src_r, dst_r)
    agg2u = _agg(movie1, src_v, dst_v)
    movie2 = _tc_call(cnt_r, agg2m, movie1, W2rl, b2rl.reshape(1, D), W2rr,
                      relu=False)
    user2 = _tc_call(cnt_v, agg2u, user1, W2vl, b2vl.reshape(1, D), W2vr,
                     relu=False)
    return (user2, movie2)


# speedup vs baseline: 1.0782x; 1.0480x over previous
import functools
import jax
import jax.numpy as jnp
from jax import lax
from jax.experimental import pallas as pl
from jax.experimental.pallas import tpu as pltpu
from jax.experimental.pallas import tpu_sc as plsc

D = 128
N_USER = 100000
N_MOVIE = 50000
E = 500000

NC, NS = 2, 16
ND_PAD = 50176
NPASS = 4
RNG_ROWS = 6272
SUB_ROWS = RNG_ROWS // NS
ZB_ROWS = 56
TRASH = RNG_ROWS

E_PAD = 524288
E_ROWS = E_PAD // 128
CHUNK_ROWS = E_ROWS // NS
SR_ROWS = 128
BLK_ROWS = 16
NBLK = SR_ROWS // BLK_ROWS
SEL_ROWS = SR_ROWS + 8

BIN_ROWS = 128
PADV = (PAD_DST := 50176) and ((50176 << 16) - (1 << 32))

CNT_PAD = 50432
CNT_SUB = CNT_PAD // NS

BM = 512


def _agg_body(table_hbm, bins_hbm, cnts_hbm, out_hbm,
              acc_sh, binst_v, cnts_v, sel_v, rows0_v, rows1_v,
              isrc0_v, idst0_v, isrc1_v, idst1_v, wb0_v, wb1_v, sem):
    c = lax.axis_index("c")
    s = lax.axis_index("s")
    zvec = jnp.zeros((16,), jnp.float32)
    ipad = jnp.full((16,), TRASH << 16, jnp.int32)
    iota = lax.iota(jnp.int32, 16)
    rows_b = (rows0_v, rows1_v)
    isrc_b = (isrc0_v, isrc1_v)
    idst_b = (idst0_v, idst1_v)

    def unpack(j, b):
        for g in range(8):
            sl = pl.ds(g * 16, 16)
            packed = sel_v[j, sl]
            isrc_b[b][sl] = jnp.bitwise_and(packed, 0xFFFF)
            idst_b[b][sl] = lax.shift_right_logical(packed, 16)

    def start_gather(b):
        return pltpu.async_copy(table_hbm.at[isrc_b[b]], rows_b[b], sem)

    def wait_gather(b):
        pltpu.make_async_copy(table_hbm.at[isrc_b[b]], rows_b[b], sem).wait()

    wb_b = (wb0_v, wb1_v)

    def zfill(r, _):
        for g in range(8):
            wb0_v[r, pl.ds(g * 16, 16)] = zvec
        return 0
    lax.fori_loop(0, ZB_ROWS, zfill, 0)

    pltpu.sync_copy(cnts_hbm.at[pl.ds(s * 32, 32)], cnts_v)

    nzb = SUB_ROWS // ZB_ROWS
    for p in range(NPASS):
        rng = p * NC + c
        lo = rng * RNG_ROWS
        for t in range(nzb):
            pltpu.async_copy(
                wb0_v, acc_sh.at[pl.ds(s * SUB_ROWS + t * ZB_ROWS, ZB_ROWS)],
                sem)
        for t in range(nzb):
            pltpu.make_async_copy(
                wb0_v, acc_sh.at[pl.ds(s * SUB_ROWS, ZB_ROWS)], sem).wait()
        plsc.subcore_barrier()

        lo16 = rng * (RNG_ROWS << 16)
        for u in range(2):
            w = 2 * s + u
            cvec = cnts_v[pl.ds(u * 16, 16)]
            cnt = jnp.max(jnp.where(iota == p, cvec, 0))
            nblk = (cnt + 2047) // 2048
            bin_base = (w * 4 + p) * BIN_ROWS

            def blk_body(b, cur):
                pltpu.sync_copy(bins_hbm.at[pl.ds(bin_base + b * 16, 16)],
                                binst_v)

                def vec_body(i, cur2):
                    r = i // 8
                    g = i % 8
                    pk = binst_v[r, pl.ds(g * 16, 16)]
                    d = lax.shift_right_logical(pk, 16)
                    ud = d - lo
                    m = plsc.bitcast(ud, jnp.uint32) < jnp.uint32(RNG_ROWS)
                    cum = plsc.cumsum(m.astype(jnp.int32))
                    pos = cur2 + cum - 1
                    row = jnp.right_shift(pos, 7)
                    col = jnp.bitwise_and(pos, 127)
                    plsc.store_scatter(sel_v, [row, col], pk - lo16, mask=m)
                    return cur2 + plsc.all_reduce_population_count(m)
                return lax.fori_loop(0, 128, vec_body, cur)
            cursor = lax.fori_loop(0, nblk, blk_body,
                                   jnp.zeros((16,), jnp.int32))

            n_sel = jnp.max(cursor)
            for k in range(8):
                pos = n_sel + k * 16 + iota
                row = jnp.right_shift(pos, 7)
                col = jnp.bitwise_and(pos, 127)
                plsc.store_scatter(sel_v, [row, col], ipad)

            nb = (n_sel + 127) // 128

            @pl.when(nb > 0)
            def _prime():
                unpack(0, 0)
                start_gather(0)

            def pair_body(jj, _):
                for b in range(2):
                    j = jj * 2 + b

                    @pl.when(j < nb)
                    def _step():
                        @pl.when(j + 1 < nb)
                        def _next():
                            unpack(j + 1, 1 - b)
                            start_gather(1 - b)
                        wait_gather(b)
                        pltpu.sync_copy(rows_b[b], acc_sh.at[idst_b[b]],
                                        add=True)
                return 0
            lax.fori_loop(0, (nb + 1) // 2, pair_body, 0)
        plsc.subcore_barrier()

        for t in range(nzb):
            b = t % 2
            off = s * SUB_ROWS + t * ZB_ROWS
            if t >= 2:
                pltpu.make_async_copy(
                    wb_b[b], out_hbm.at[pl.ds(lo, ZB_ROWS)], sem).wait()
            pltpu.sync_copy(acc_sh.at[pl.ds(off, ZB_ROWS)], wb_b[b])
            pltpu.async_copy(wb_b[b], out_hbm.at[pl.ds(lo + off, ZB_ROWS)],
                             sem)
        for b in range(2):
            pltpu.make_async_copy(
                wb_b[b], out_hbm.at[pl.ds(lo, ZB_ROWS)], sem).wait()
        plsc.subcore_barrier()
        if p + 1 < NPASS:
            lax.fori_loop(0, ZB_ROWS, zfill, 0)


def _bin_body(src_hbm, dst_hbm, bins_hbm, cnts_hbm,
              src_v, dst_v, sel0, sel1, sel2, sel3, cnts_v, sem):
    c = lax.axis_index("c")
    s = lax.axis_index("s")
    w = s * NC + c
    sels = (sel0, sel1, sel2, sel3)
    iota = lax.iota(jnp.int32, 16)
    padv = jnp.full((16,), PADV, jnp.int32)

    def pfill(r, _):
        for q in range(4):
            for g in range(8):
                sels[q][r, pl.ds(g * 16, 16)] = padv
        return 0
    lax.fori_loop(0, BIN_ROWS, pfill, 0)

    cursors = (jnp.zeros((16,), jnp.int32),) * 4
    for b in range(8):
        base = w * 128 + b * 16
        pltpu.async_copy(src_hbm.at[pl.ds(base, 16)], src_v, sem)
        pltpu.async_copy(dst_hbm.at[pl.ds(base, 16)], dst_v, sem)
        pltpu.make_async_copy(src_hbm.at[pl.ds(base, 16)], src_v, sem).wait()
        pltpu.make_async_copy(dst_hbm.at[pl.ds(base, 16)], dst_v, sem).wait()

        def vec_body(i, curs):
            r = i // 8
            g = i % 8
            sl = pl.ds(g * 16, 16)
            d = dst_v[r, sl]
            sr = src_v[r, sl]
            pid = ((d >= 2 * RNG_ROWS).astype(jnp.int32)
                   + (d >= 4 * RNG_ROWS).astype(jnp.int32)
                   + (d >= 6 * RNG_ROWS).astype(jnp.int32))
            pk = jnp.bitwise_or(
                sr, lax.shift_left(d, jnp.full((16,), 16, jnp.int32)))
            new = []
            for q in range(4):
                mq = pid == q
                cum = plsc.cumsum(mq.astype(jnp.int32))
                pos = curs[q] + cum - 1
                row = jnp.right_shift(pos, 7)
                col = jnp.bitwise_and(pos, 127)
                plsc.store_scatter(sels[q], [row, col], pk, mask=mq)
                new.append(curs[q] + plsc.all_reduce_population_count(mq))
            return tuple(new)
        cursors = lax.fori_loop(0, 128, vec_body, cursors)

    cv = jnp.zeros((16,), jnp.int32)
    for q in range(4):
        cv = jnp.where(iota == q, cursors[q], cv)
    cnts_v[...] = cv
    pltpu.sync_copy(cnts_v, cnts_hbm.at[pl.ds(w * 16, 16)])
    for q in range(4):
        pltpu.sync_copy(sels[q],
                        bins_hbm.at[pl.ds((w * 4 + q) * BIN_ROWS, BIN_ROWS)])


@jax.jit
def _bin(src2d, dst2d):
    return pl.kernel(
        _bin_body,
        out_type=(jax.ShapeDtypeStruct((NC * NS * 4 * BIN_ROWS, 128),
                                       jnp.int32),
                  jax.ShapeDtypeStruct((NC * NS * 16,), jnp.int32)),
        mesh=_SC_MESH,
        scratch_types=[
            pltpu.VMEM((16, 128), jnp.int32),
            pltpu.VMEM((16, 128), jnp.int32),
            pltpu.VMEM((BIN_ROWS, 128), jnp.int32),
            pltpu.VMEM((BIN_ROWS, 128), jnp.int32),
            pltpu.VMEM((BIN_ROWS, 128), jnp.int32),
            pltpu.VMEM((BIN_ROWS, 128), jnp.int32),
            pltpu.VMEM((16,), jnp.int32),
            pltpu.SemaphoreType.DMA,
        ],
        compiler_params=pltpu.CompilerParams(needs_layout_passes=False),
    )(src2d, dst2d)


def _cnt_body(dst_hbm, out_hbm, sh_cnt, dst_v, ones_v, stage_v):
    c = lax.axis_index("c")
    s = lax.axis_index("s")
    wid = s * NC + c
    izero = jnp.zeros((16,), jnp.int32)
    for g in range(8):
        ones_v[pl.ds(g * 16, 16)] = jnp.ones((16,), jnp.int32)
    for k in range(CNT_SUB // 16):
        stage_v[pl.ds(k * 16, 16)] = izero
    pltpu.sync_copy(stage_v, sh_cnt.at[pl.ds(s * CNT_SUB, CNT_SUB)])
    plsc.subcore_barrier()

    rows_per_tile = E_ROWS // (NC * NS)
    pltpu.sync_copy(dst_hbm.at[pl.ds(wid * rows_per_tile, rows_per_tile)],
                    dst_v)

    def row_body(r, _):
        pltpu.sync_copy(ones_v, sh_cnt.at[dst_v.at[r]], add=True)
        return 0
    lax.fori_loop(0, rows_per_tile, row_body, 0)
    plsc.subcore_barrier()
    pltpu.sync_copy(sh_cnt.at[pl.ds(s * CNT_SUB, CNT_SUB)], stage_v)
    pltpu.sync_copy(stage_v,
                    out_hbm.at[pl.ds(c * CNT_PAD + s * CNT_SUB, CNT_SUB)])


_SC_MESH = plsc.VectorSubcoreMesh(core_axis_name="c", subcore_axis_name="s")


@jax.jit
def _agg(table, bins, cnts):
    return pl.kernel(
        _agg_body,
        out_type=jax.ShapeDtypeStruct((ND_PAD, D), jnp.float32),
        mesh=_SC_MESH,
        scratch_types=[
            pltpu.VMEM_SHARED((RNG_ROWS + 16, D), jnp.float32),
            pltpu.VMEM((16, 128), jnp.int32),
            pltpu.VMEM((32,), jnp.int32),
            pltpu.VMEM((SEL_ROWS, 128), jnp.int32),
            pltpu.VMEM((128, D), jnp.float32),
            pltpu.VMEM((128, D), jnp.float32),
            pltpu.VMEM((128,), jnp.int32),
            pltpu.VMEM((128,), jnp.int32),
            pltpu.VMEM((128,), jnp.int32),
            pltpu.VMEM((128,), jnp.int32),
            pltpu.VMEM((ZB_ROWS, D), jnp.float32),
            pltpu.VMEM((ZB_ROWS, D), jnp.float32),
            pltpu.SemaphoreType.DMA,
        ],
        compiler_params=pltpu.CompilerParams(needs_layout_passes=False),
    )(table, bins, cnts)


@jax.jit
def _count(dst2d):
    return pl.kernel(
        _cnt_body,
        out_type=jax.ShapeDtypeStruct((NC * CNT_PAD,), jnp.int32),
        mesh=_SC_MESH,
        scratch_types=[
            pltpu.VMEM_SHARED((CNT_PAD,), jnp.int32),
            pltpu.VMEM((E_ROWS // (NC * NS), 128), jnp.int32),
            pltpu.VMEM((128,), jnp.int32),
            pltpu.VMEM((CNT_SUB,), jnp.int32),
        ],
    )(dst2d)


def _tc_body(cnt_ref, agg_ref, x_ref, wl_ref, bl_ref, wr_ref, o_ref,
             *, relu, nblk_agg):
    i = pl.program_id(0)
    valid = (i < nblk_agg).astype(jnp.float32)
    cnt = (cnt_ref[0, :] + cnt_ref[1, :]).astype(jnp.float32)
    inv = valid / jnp.maximum(cnt, 1.0)
    agg = agg_ref[...] * inv[:, None]
    acc = jnp.dot(agg, wl_ref[...], preferred_element_type=jnp.float32)
    acc = acc + jnp.dot(x_ref[...], wr_ref[...],
                        preferred_element_type=jnp.float32)
    acc = acc + bl_ref[...]
    if relu:
        acc = jnp.maximum(acc, 0.0)
    o_ref[...] = acc


def _tc_call(cnt, agg, x, Wl, bl, Wr, relu):
    n = x.shape[0]
    nblk_agg = ND_PAD // BM
    grid = pl.cdiv(n, BM)
    clamp = lambda i: jnp.minimum(i, nblk_agg - 1)
    return pl.pallas_call(
        functools.partial(_tc_body, relu=relu, nblk_agg=nblk_agg),
        grid=(grid,),
        in_specs=[
            pl.BlockSpec((2, BM), lambda i: (0, clamp(i))),
            pl.BlockSpec((BM, D), lambda i: (clamp(i), 0)),
            pl.BlockSpec((BM, D), lambda i: (i, 0)),
            pl.BlockSpec((D, D), lambda i: (0, 0)),
            pl.BlockSpec((1, D), lambda i: (0, 0)),
            pl.BlockSpec((D, D), lambda i: (0, 0)),
        ],
        out_specs=pl.BlockSpec((BM, D), lambda i: (i, 0)),
        out_shape=jax.ShapeDtypeStruct((n, D), jnp.float32),
    )(cnt, agg, x, Wl, bl, Wr)


def _pad_edges(edge_index):
    src = jnp.concatenate(
        [edge_index[0], jnp.zeros((E_PAD - E,), jnp.int32)]).reshape(E_ROWS, 128)
    dst = jnp.concatenate(
        [edge_index[1], jnp.full((E_PAD - E,), PAD_DST, jnp.int32)]
    ).reshape(E_ROWS, 128)
    return src, dst


def kernel(x_user, x_movie, edge_index_rates, edge_index_rev_rates,
           W1rl, b1rl, W1rr, W1vl, b1vl, W1vr,
           W2rl, b2rl, W2rr, W2vl, b2vl, W2vr):
    src_r, dst_r = _pad_edges(edge_index_rates)
    src_v, dst_v = _pad_edges(edge_index_rev_rates)
    cnt_r = _count(dst_r).reshape(NC, CNT_PAD)
    cnt_v = _count(dst_v).reshape(NC, CNT_PAD)

    bins_r, bcnt_r = _bin(src_r, dst_r)
    bins_v, bcnt_v = _bin(src_v, dst_v)

    agg1m = _agg(x_user, bins_r, bcnt_r)
    agg1u = _agg(x_movie, bins_v, bcnt_v)
    movie1 = _tc_call(cnt_r, agg1m, x_movie, W1rl, b1rl.reshape(1, D), W1rr,
                      relu=True)
    user1 = _tc_call(cnt_v, agg1u, x_user, W1vl, b1vl.reshape(1, D), W1vr,
                     relu=True)

    agg2m = _agg(user1, bins_r, bcnt_r)
    agg2u = _agg(movie1, bins_v, bcnt_v)
    movie2 = _tc_call(cnt_r, agg2m, movie1, W2rl, b2rl.reshape(1, D), W2rr,
                      relu=False)
    user2 = _tc_call(cnt_v, agg2u, user1, W2vl, b2vl.reshape(1, D), W2vr,
                     relu=False)
    return (user2, movie2)


# speedup vs baseline: 1.0824x; 1.0039x over previous
import functools
import jax
import jax.numpy as jnp
from jax import lax
from jax.experimental import pallas as pl
from jax.experimental.pallas import tpu as pltpu
from jax.experimental.pallas import tpu_sc as plsc

D = 128
N_USER = 100000
N_MOVIE = 50000
E = 500000

NC, NS = 2, 16
ND_PAD = 50176
NPASS = 4
RNG_ROWS = 6272
SUB_ROWS = RNG_ROWS // NS
ZB_ROWS = 56
TRASH = RNG_ROWS

E_PAD = 524288
E_ROWS = E_PAD // 128
CHUNK_ROWS = E_ROWS // NS
SR_ROWS = 128
BLK_ROWS = 16
NBLK = SR_ROWS // BLK_ROWS
SEL_ROWS = SR_ROWS + 8

BIN_ROWS = 128
PADV = (PAD_DST := 50176) and ((50176 << 16) - (1 << 32))

CNT_PAD = 50432
CNT_SUB = CNT_PAD // NS

BM = 512


def _agg_body(table_hbm, bins_hbm, cnts_hbm, out_hbm,
              acc_sh, binst0_v, binst1_v, cnts_v, sel_v, rows0_v, rows1_v,
              isrc0_v, idst0_v, isrc1_v, idst1_v, wb0_v, wb1_v, sem):
    c = lax.axis_index("c")
    s = lax.axis_index("s")
    zvec = jnp.zeros((16,), jnp.float32)
    ipad = jnp.full((16,), TRASH << 16, jnp.int32)
    iota = lax.iota(jnp.int32, 16)
    rows_b = (rows0_v, rows1_v)
    isrc_b = (isrc0_v, isrc1_v)
    idst_b = (idst0_v, idst1_v)

    def unpack(j, b):
        for g in range(8):
            sl = pl.ds(g * 16, 16)
            packed = sel_v[j, sl]
            isrc_b[b][sl] = jnp.bitwise_and(packed, 0xFFFF)
            idst_b[b][sl] = lax.shift_right_logical(packed, 16)

    def start_gather(b):
        return pltpu.async_copy(table_hbm.at[isrc_b[b]], rows_b[b], sem)

    def wait_gather(b):
        pltpu.make_async_copy(table_hbm.at[isrc_b[b]], rows_b[b], sem).wait()

    wb_b = (wb0_v, wb1_v)

    def zfill(r, _):
        for g in range(8):
            wb0_v[r, pl.ds(g * 16, 16)] = zvec
        return 0
    lax.fori_loop(0, ZB_ROWS, zfill, 0)

    pltpu.sync_copy(cnts_hbm.at[pl.ds(s * 32, 32)], cnts_v)

    nzb = SUB_ROWS // ZB_ROWS
    for p in range(NPASS):
        rng = p * NC + c
        lo = rng * RNG_ROWS
        for t in range(nzb):
            pltpu.async_copy(
                wb0_v, acc_sh.at[pl.ds(s * SUB_ROWS + t * ZB_ROWS, ZB_ROWS)],
                sem)
        for t in range(nzb):
            pltpu.make_async_copy(
                wb0_v, acc_sh.at[pl.ds(s * SUB_ROWS, ZB_ROWS)], sem).wait()
        plsc.subcore_barrier()

        lo16 = rng * (RNG_ROWS << 16)
        for u in range(2):
            w = 2 * s + u
            cvec = cnts_v[pl.ds(u * 16, 16)]
            cnt = jnp.max(jnp.where(iota == p, cvec, 0))
            nblk = (cnt + 2047) // 2048
            bin_base = (w * 4 + p) * BIN_ROWS
            binst_b = (binst0_v, binst1_v)

            def bin_load(b, h):
                pltpu.async_copy(bins_hbm.at[pl.ds(bin_base + b * 16, 16)],
                                 binst_b[h], sem)

            def bin_wait(h):
                pltpu.make_async_copy(bins_hbm.at[pl.ds(bin_base, 16)],
                                      binst_b[h], sem).wait()

            @pl.when(nblk > 0)
            def _bprime():
                bin_load(0, 0)

            def blk2_body(bb, cur):
                for h in range(2):
                    b = bb * 2 + h

                    def scan_blk(cur_in):
                        bin_wait(h)

                        @pl.when(b + 1 < nblk)
                        def _pref():
                            bin_load(b + 1, 1 - h)

                        def vec_body(i, cur2):
                            r = i // 8
                            g = i % 8
                            pk = binst_b[h][r, pl.ds(g * 16, 16)]
                            d = lax.shift_right_logical(pk, 16)
                            ud = d - lo
                            m = (plsc.bitcast(ud, jnp.uint32)
                                 < jnp.uint32(RNG_ROWS))
                            cum = plsc.cumsum(m.astype(jnp.int32))
                            pos = cur2 + cum - 1
                            row = jnp.right_shift(pos, 7)
                            col = jnp.bitwise_and(pos, 127)
                            plsc.store_scatter(sel_v, [row, col], pk - lo16,
                                               mask=m)
                            return cur2 + plsc.all_reduce_population_count(m)
                        return lax.fori_loop(0, 128, vec_body, cur_in)
                    cur = lax.cond(b < nblk, scan_blk, lambda x: x, cur)
                return cur
            cursor = lax.fori_loop(0, (nblk + 1) // 2, blk2_body,
                                   jnp.zeros((16,), jnp.int32))

            n_sel = jnp.max(cursor)
            for k in range(8):
                pos = n_sel + k * 16 + iota
                row = jnp.right_shift(pos, 7)
                col = jnp.bitwise_and(pos, 127)
                plsc.store_scatter(sel_v, [row, col], ipad)

            nb = (n_sel + 127) // 128

            @pl.when(nb > 0)
            def _prime():
                unpack(0, 0)
                start_gather(0)

            def pair_body(jj, _):
                for b in range(2):
                    j = jj * 2 + b

                    @pl.when(j < nb)
                    def _step():
                        @pl.when(j + 1 < nb)
                        def _next():
                            unpack(j + 1, 1 - b)
                            start_gather(1 - b)
                        wait_gather(b)
                        pltpu.sync_copy(rows_b[b], acc_sh.at[idst_b[b]],
                                        add=True)
                return 0
            lax.fori_loop(0, (nb + 1) // 2, pair_body, 0)
        plsc.subcore_barrier()

        for t in range(nzb):
            b = t % 2
            off = s * SUB_ROWS + t * ZB_ROWS
            if t >= 2:
                pltpu.make_async_copy(
                    wb_b[b], out_hbm.at[pl.ds(lo, ZB_ROWS)], sem).wait()
            pltpu.sync_copy(acc_sh.at[pl.ds(off, ZB_ROWS)], wb_b[b])
            pltpu.async_copy(wb_b[b], out_hbm.at[pl.ds(lo + off, ZB_ROWS)],
                             sem)
        for b in range(2):
            pltpu.make_async_copy(
                wb_b[b], out_hbm.at[pl.ds(lo, ZB_ROWS)], sem).wait()
        plsc.subcore_barrier()
        if p + 1 < NPASS:
            lax.fori_loop(0, ZB_ROWS, zfill, 0)


def _bin_body(src_hbm, dst_hbm, bins_hbm, cnts_hbm,
              src_v, dst_v, sel0, sel1, sel2, sel3, cnts_v, sem):
    c = lax.axis_index("c")
    s = lax.axis_index("s")
    w = s * NC + c
    sels = (sel0, sel1, sel2, sel3)
    iota = lax.iota(jnp.int32, 16)
    padv = jnp.full((16,), PADV, jnp.int32)

    def pfill(r, _):
        for q in range(4):
            for g in range(8):
                sels[q][r, pl.ds(g * 16, 16)] = padv
        return 0
    lax.fori_loop(0, BIN_ROWS, pfill, 0)

    cursors = (jnp.zeros((16,), jnp.int32),) * 4
    for b in range(8):
        base = w * 128 + b * 16
        pltpu.async_copy(src_hbm.at[pl.ds(base, 16)], src_v, sem)
        pltpu.async_copy(dst_hbm.at[pl.ds(base, 16)], dst_v, sem)
        pltpu.make_async_copy(src_hbm.at[pl.ds(base, 16)], src_v, sem).wait()
        pltpu.make_async_copy(dst_hbm.at[pl.ds(base, 16)], dst_v, sem).wait()

        def vec_body(i, curs):
            r = i // 8
            g = i % 8
            sl = pl.ds(g * 16, 16)
            d = dst_v[r, sl]
            sr = src_v[r, sl]
            pid = ((d >= 2 * RNG_ROWS).astype(jnp.int32)
                   + (d >= 4 * RNG_ROWS).astype(jnp.int32)
                   + (d >= 6 * RNG_ROWS).astype(jnp.int32))
            pk = jnp.bitwise_or(
                sr, lax.shift_left(d, jnp.full((16,), 16, jnp.int32)))
            new = []
            for q in range(4):
                mq = pid == q
                cum = plsc.cumsum(mq.astype(jnp.int32))
                pos = curs[q] + cum - 1
                row = jnp.right_shift(pos, 7)
                col = jnp.bitwise_and(pos, 127)
                plsc.store_scatter(sels[q], [row, col], pk, mask=mq)
                new.append(curs[q] + plsc.all_reduce_population_count(mq))
            return tuple(new)
        cursors = lax.fori_loop(0, 128, vec_body, cursors)

    cv = jnp.zeros((16,), jnp.int32)
    for q in range(4):
        cv = jnp.where(iota == q, cursors[q], cv)
    cnts_v[...] = cv
    pltpu.sync_copy(cnts_v, cnts_hbm.at[pl.ds(w * 16, 16)])
    for q in range(4):
        pltpu.sync_copy(sels[q],
                        bins_hbm.at[pl.ds((w * 4 + q) * BIN_ROWS, BIN_ROWS)])


@jax.jit
def _bin(src2d, dst2d):
    return pl.kernel(
        _bin_body,
        out_type=(jax.ShapeDtypeStruct((NC * NS * 4 * BIN_ROWS, 128),
                                       jnp.int32),
                  jax.ShapeDtypeStruct((NC * NS * 16,), jnp.int32)),
        mesh=_SC_MESH,
        scratch_types=[
            pltpu.VMEM((16, 128), jnp.int32),
            pltpu.VMEM((16, 128), jnp.int32),
            pltpu.VMEM((BIN_ROWS, 128), jnp.int32),
            pltpu.VMEM((BIN_ROWS, 128), jnp.int32),
            pltpu.VMEM((BIN_ROWS, 128), jnp.int32),
            pltpu.VMEM((BIN_ROWS, 128), jnp.int32),
            pltpu.VMEM((16,), jnp.int32),
            pltpu.SemaphoreType.DMA,
        ],
        compiler_params=pltpu.CompilerParams(needs_layout_passes=False),
    )(src2d, dst2d)


def _cnt_body(dst_hbm, out_hbm, sh_cnt, dst_v, ones_v, stage_v):
    c = lax.axis_index("c")
    s = lax.axis_index("s")
    wid = s * NC + c
    izero = jnp.zeros((16,), jnp.int32)
    for g in range(8):
        ones_v[pl.ds(g * 16, 16)] = jnp.ones((16,), jnp.int32)
    for k in range(CNT_SUB // 16):
        stage_v[pl.ds(k * 16, 16)] = izero
    pltpu.sync_copy(stage_v, sh_cnt.at[pl.ds(s * CNT_SUB, CNT_SUB)])
    plsc.subcore_barrier()

    rows_per_tile = E_ROWS // (NC * NS)
    pltpu.sync_copy(dst_hbm.at[pl.ds(wid * rows_per_tile, rows_per_tile)],
                    dst_v)

    def row_body(r, _):
        pltpu.sync_copy(ones_v, sh_cnt.at[dst_v.at[r]], add=True)
        return 0
    lax.fori_loop(0, rows_per_tile, row_body, 0)
    plsc.subcore_barrier()
    pltpu.sync_copy(sh_cnt.at[pl.ds(s * CNT_SUB, CNT_SUB)], stage_v)
    pltpu.sync_copy(stage_v,
                    out_hbm.at[pl.ds(c * CNT_PAD + s * CNT_SUB, CNT_SUB)])


_SC_MESH = plsc.VectorSubcoreMesh(core_axis_name="c", subcore_axis_name="s")


@jax.jit
def _agg(table, bins, cnts):
    return pl.kernel(
        _agg_body,
        out_type=jax.ShapeDtypeStruct((ND_PAD, D), jnp.float32),
        mesh=_SC_MESH,
        scratch_types=[
            pltpu.VMEM_SHARED((RNG_ROWS + 16, D), jnp.float32),
            pltpu.VMEM((16, 128), jnp.int32),
            pltpu.VMEM((16, 128), jnp.int32),
            pltpu.VMEM((32,), jnp.int32),
            pltpu.VMEM((SEL_ROWS, 128), jnp.int32),
            pltpu.VMEM((128, D), jnp.float32),
            pltpu.VMEM((128, D), jnp.float32),
            pltpu.VMEM((128,), jnp.int32),
            pltpu.VMEM((128,), jnp.int32),
            pltpu.VMEM((128,), jnp.int32),
            pltpu.VMEM((128,), jnp.int32),
            pltpu.VMEM((ZB_ROWS, D), jnp.float32),
            pltpu.VMEM((ZB_ROWS, D), jnp.float32),
            pltpu.SemaphoreType.DMA,
        ],
        compiler_params=pltpu.CompilerParams(needs_layout_passes=False),
    )(table, bins, cnts)


@jax.jit
def _count(dst2d):
    return pl.kernel(
        _cnt_body,
        out_type=jax.ShapeDtypeStruct((NC * CNT_PAD,), jnp.int32),
        mesh=_SC_MESH,
        scratch_types=[
            pltpu.VMEM_SHARED((CNT_PAD,), jnp.int32),
            pltpu.VMEM((E_ROWS // (NC * NS), 128), jnp.int32),
            pltpu.VMEM((128,), jnp.int32),
            pltpu.VMEM((CNT_SUB,), jnp.int32),
        ],
    )(dst2d)


def _tc_body(cnt_ref, agg_ref, x_ref, wl_ref, bl_ref, wr_ref, o_ref,
             *, relu, nblk_agg):
    i = pl.program_id(0)
    valid = (i < nblk_agg).astype(jnp.float32)
    cnt = (cnt_ref[0, :] + cnt_ref[1, :]).astype(jnp.float32)
    inv = valid / jnp.maximum(cnt, 1.0)
    agg = agg_ref[...] * inv[:, None]
    acc = jnp.dot(agg, wl_ref[...], preferred_element_type=jnp.float32)
    acc = acc + jnp.dot(x_ref[...], wr_ref[...],
                        preferred_element_type=jnp.float32)
    acc = acc + bl_ref[...]
    if relu:
        acc = jnp.maximum(acc, 0.0)
    o_ref[...] = acc


def _tc_call(cnt, agg, x, Wl, bl, Wr, relu):
    n = x.shape[0]
    nblk_agg = ND_PAD // BM
    grid = pl.cdiv(n, BM)
    clamp = lambda i: jnp.minimum(i, nblk_agg - 1)
    return pl.pallas_call(
        functools.partial(_tc_body, relu=relu, nblk_agg=nblk_agg),
        grid=(grid,),
        in_specs=[
            pl.BlockSpec((2, BM), lambda i: (0, clamp(i))),
            pl.BlockSpec((BM, D), lambda i: (clamp(i), 0)),
            pl.BlockSpec((BM, D), lambda i: (i, 0)),
            pl.BlockSpec((D, D), lambda i: (0, 0)),
            pl.BlockSpec((1, D), lambda i: (0, 0)),
            pl.BlockSpec((D, D), lambda i: (0, 0)),
        ],
        out_specs=pl.BlockSpec((BM, D), lambda i: (i, 0)),
        out_shape=jax.ShapeDtypeStruct((n, D), jnp.float32),
    )(cnt, agg, x, Wl, bl, Wr)


def _pad_edges(edge_index):
    src = jnp.concatenate(
        [edge_index[0], jnp.zeros((E_PAD - E,), jnp.int32)]).reshape(E_ROWS, 128)
    dst = jnp.concatenate(
        [edge_index[1], jnp.full((E_PAD - E,), PAD_DST, jnp.int32)]
    ).reshape(E_ROWS, 128)
    return src, dst


def kernel(x_user, x_movie, edge_index_rates, edge_index_rev_rates,
           W1rl, b1rl, W1rr, W1vl, b1vl, W1vr,
           W2rl, b2rl, W2rr, W2vl, b2vl, W2vr):
    src_r, dst_r = _pad_edges(edge_index_rates)
    src_v, dst_v = _pad_edges(edge_index_rev_rates)
    cnt_r = _count(dst_r).reshape(NC, CNT_PAD)
    cnt_v = _count(dst_v).reshape(NC, CNT_PAD)

    bins_r, bcnt_r = _bin(src_r, dst_r)
    bins_v, bcnt_v = _bin(src_v, dst_v)

    agg1m = _agg(x_user, bins_r, bcnt_r)
    agg1u = _agg(x_movie, bins_v, bcnt_v)
    movie1 = _tc_call(cnt_r, agg1m, x_movie, W1rl, b1rl.reshape(1, D), W1rr,
                      relu=True)
    user1 = _tc_call(cnt_v, agg1u, x_user, W1vl, b1vl.reshape(1, D), W1vr,
                     relu=True)

    agg2m = _agg(user1, bins_r, bcnt_r)
    agg2u = _agg(movie1, bins_v, bcnt_v)
    movie2 = _tc_call(cnt_r, agg2m, movie1, W2rl, b2rl.reshape(1, D), W2rr,
                      relu=False)
    user2 = _tc_call(cnt_v, agg2u, user1, W2vl, b2vl.reshape(1, D), W2vr,
                     relu=False)
    return (user2, movie2)


# speedup vs baseline: 1.1106x; 1.0261x over previous
import functools
import jax
import jax.numpy as jnp
from jax import lax
from jax.experimental import pallas as pl
from jax.experimental.pallas import tpu as pltpu
from jax.experimental.pallas import tpu_sc as plsc

D = 128
N_USER = 100000
N_MOVIE = 50000
E = 500000

NC, NS = 2, 16
ND_PAD = 50176
NPASS = 4
RNG_ROWS = 6272
SUB_ROWS = RNG_ROWS // NS
ZB_ROWS = 56
TRASH = RNG_ROWS

E_PAD = 524288
E_ROWS = E_PAD // 128
CHUNK_ROWS = E_ROWS // NS
SR_ROWS = 128
BLK_ROWS = 16
NBLK = SR_ROWS // BLK_ROWS
SEL_ROWS = SR_ROWS + 8

BIN_ROWS = 128
PADV = (PAD_DST := 50176) and ((50176 << 16) - (1 << 32))

CNT_PAD = 50432
CNT_SUB = CNT_PAD // NS

BM = 1024


def _agg_body(table_hbm, bins_hbm, cnts_hbm, out_hbm,
              acc_sh, binst0_v, binst1_v, cnts_v, sel_v, rows0_v, rows1_v,
              isrc0_v, idst0_v, isrc1_v, idst1_v, wb0_v, wb1_v, sem):
    c = lax.axis_index("c")
    s = lax.axis_index("s")
    zvec = jnp.zeros((16,), jnp.float32)
    ipad = jnp.full((16,), TRASH << 16, jnp.int32)
    iota = lax.iota(jnp.int32, 16)
    rows_b = (rows0_v, rows1_v)
    isrc_b = (isrc0_v, isrc1_v)
    idst_b = (idst0_v, idst1_v)

    def unpack(j, b):
        for g in range(8):
            sl = pl.ds(g * 16, 16)
            packed = sel_v[j, sl]
            isrc_b[b][sl] = jnp.bitwise_and(packed, 0xFFFF)
            idst_b[b][sl] = lax.shift_right_logical(packed, 16)

    def start_gather(b):
        return pltpu.async_copy(table_hbm.at[isrc_b[b]], rows_b[b], sem)

    def wait_gather(b):
        pltpu.make_async_copy(table_hbm.at[isrc_b[b]], rows_b[b], sem).wait()

    wb_b = (wb0_v, wb1_v)

    def zfill(r, _):
        for g in range(8):
            wb0_v[r, pl.ds(g * 16, 16)] = zvec
        return 0
    lax.fori_loop(0, ZB_ROWS, zfill, 0)

    pltpu.sync_copy(cnts_hbm.at[pl.ds(s * 32, 32)], cnts_v)

    nzb = SUB_ROWS // ZB_ROWS
    for p in range(NPASS):
        rng = p * NC + c
        lo = rng * RNG_ROWS
        for t in range(nzb):
            pltpu.async_copy(
                wb0_v, acc_sh.at[pl.ds(s * SUB_ROWS + t * ZB_ROWS, ZB_ROWS)],
                sem)
        for t in range(nzb):
            pltpu.make_async_copy(
                wb0_v, acc_sh.at[pl.ds(s * SUB_ROWS, ZB_ROWS)], sem).wait()
        plsc.subcore_barrier()

        lo16 = rng * (RNG_ROWS << 16)
        for u in range(2):
            w = 2 * s + u
            cvec = cnts_v[pl.ds(u * 16, 16)]
            cnt = jnp.max(jnp.where(iota == p, cvec, 0))
            nblk = (cnt + 2047) // 2048
            bin_base = (w * 4 + p) * BIN_ROWS
            binst_b = (binst0_v, binst1_v)

            def bin_load(b, h):
                pltpu.async_copy(bins_hbm.at[pl.ds(bin_base + b * 16, 16)],
                                 binst_b[h], sem)

            def bin_wait(h):
                pltpu.make_async_copy(bins_hbm.at[pl.ds(bin_base, 16)],
                                      binst_b[h], sem).wait()

            @pl.when(nblk > 0)
            def _bprime():
                bin_load(0, 0)

            def blk2_body(bb, cur):
                for h in range(2):
                    b = bb * 2 + h

                    def scan_blk(cur_in):
                        bin_wait(h)

                        @pl.when(b + 1 < nblk)
                        def _pref():
                            bin_load(b + 1, 1 - h)

                        def vec_body(i, cur2):
                            r = i // 8
                            g = i % 8
                            pk = binst_b[h][r, pl.ds(g * 16, 16)]
                            d = lax.shift_right_logical(pk, 16)
                            ud = d - lo
                            m = (plsc.bitcast(ud, jnp.uint32)
                                 < jnp.uint32(RNG_ROWS))
                            cum = plsc.cumsum(m.astype(jnp.int32))
                            pos = cur2 + cum - 1
                            row = jnp.right_shift(pos, 7)
                            col = jnp.bitwise_and(pos, 127)
                            plsc.store_scatter(sel_v, [row, col], pk - lo16,
                                               mask=m)
                            return cur2 + plsc.all_reduce_population_count(m)
                        return lax.fori_loop(0, 128, vec_body, cur_in)
                    cur = lax.cond(b < nblk, scan_blk, lambda x: x, cur)
                return cur
            cursor = lax.fori_loop(0, (nblk + 1) // 2, blk2_body,
                                   jnp.zeros((16,), jnp.int32))

            n_sel = jnp.max(cursor)
            for k in range(8):
                pos = n_sel + k * 16 + iota
                row = jnp.right_shift(pos, 7)
                col = jnp.bitwise_and(pos, 127)
                plsc.store_scatter(sel_v, [row, col], ipad)

            nb = (n_sel + 127) // 128

            @pl.when(nb > 0)
            def _prime():
                unpack(0, 0)
                start_gather(0)

            def pair_body(jj, _):
                for b in range(2):
                    j = jj * 2 + b

                    @pl.when(j < nb)
                    def _step():
                        @pl.when(j + 1 < nb)
                        def _next():
                            unpack(j + 1, 1 - b)
                            start_gather(1 - b)
                        wait_gather(b)
                        pltpu.sync_copy(rows_b[b], acc_sh.at[idst_b[b]],
                                        add=True)
                return 0
            lax.fori_loop(0, (nb + 1) // 2, pair_body, 0)
        plsc.subcore_barrier()

        for t in range(nzb):
            b = t % 2
            off = s * SUB_ROWS + t * ZB_ROWS
            if t >= 2:
                pltpu.make_async_copy(
                    wb_b[b], out_hbm.at[pl.ds(lo, ZB_ROWS)], sem).wait()
            pltpu.sync_copy(acc_sh.at[pl.ds(off, ZB_ROWS)], wb_b[b])
            pltpu.async_copy(wb_b[b], out_hbm.at[pl.ds(lo + off, ZB_ROWS)],
                             sem)
        for b in range(2):
            pltpu.make_async_copy(
                wb_b[b], out_hbm.at[pl.ds(lo, ZB_ROWS)], sem).wait()
        plsc.subcore_barrier()
        if p + 1 < NPASS:
            lax.fori_loop(0, ZB_ROWS, zfill, 0)


def _bin_body(src_hbm, dst_hbm, bins_hbm, cnts_hbm,
              src_v, dst_v, sel0, sel1, sel2, sel3, cnts_v, sem):
    c = lax.axis_index("c")
    s = lax.axis_index("s")
    w = s * NC + c
    sels = (sel0, sel1, sel2, sel3)
    iota = lax.iota(jnp.int32, 16)
    padv = jnp.full((16,), PADV, jnp.int32)

    def pfill(r, _):
        for q in range(4):
            for g in range(8):
                sels[q][r, pl.ds(g * 16, 16)] = padv
        return 0
    lax.fori_loop(0, BIN_ROWS, pfill, 0)

    cursors = (jnp.zeros((16,), jnp.int32),) * 4
    for b in range(8):
        base = w * 128 + b * 16
        pltpu.async_copy(src_hbm.at[pl.ds(base, 16)], src_v, sem)
        pltpu.async_copy(dst_hbm.at[pl.ds(base, 16)], dst_v, sem)
        pltpu.make_async_copy(src_hbm.at[pl.ds(base, 16)], src_v, sem).wait()
        pltpu.make_async_copy(dst_hbm.at[pl.ds(base, 16)], dst_v, sem).wait()

        def vec_body(i, curs):
            r = i // 8
            g = i % 8
            sl = pl.ds(g * 16, 16)
            d = dst_v[r, sl]
            sr = src_v[r, sl]
            pid = ((d >= 2 * RNG_ROWS).astype(jnp.int32)
                   + (d >= 4 * RNG_ROWS).astype(jnp.int32)
                   + (d >= 6 * RNG_ROWS).astype(jnp.int32))
            pk = jnp.bitwise_or(
                sr, lax.shift_left(d, jnp.full((16,), 16, jnp.int32)))
            new = []
            for q in range(4):
                mq = pid == q
                cum = plsc.cumsum(mq.astype(jnp.int32))
                pos = curs[q] + cum - 1
                row = jnp.right_shift(pos, 7)
                col = jnp.bitwise_and(pos, 127)
                plsc.store_scatter(sels[q], [row, col], pk, mask=mq)
                new.append(curs[q] + plsc.all_reduce_population_count(mq))
            return tuple(new)
        cursors = lax.fori_loop(0, 128, vec_body, cursors)

    cv = jnp.zeros((16,), jnp.int32)
    for q in range(4):
        cv = jnp.where(iota == q, cursors[q], cv)
    cnts_v[...] = cv
    pltpu.sync_copy(cnts_v, cnts_hbm.at[pl.ds(w * 16, 16)])
    for q in range(4):
        pltpu.sync_copy(sels[q],
                        bins_hbm.at[pl.ds((w * 4 + q) * BIN_ROWS, BIN_ROWS)])


@jax.jit
def _bin(src2d, dst2d):
    return pl.kernel(
        _bin_body,
        out_type=(jax.ShapeDtypeStruct((NC * NS * 4 * BIN_ROWS, 128),
                                       jnp.int32),
                  jax.ShapeDtypeStruct((NC * NS * 16,), jnp.int32)),
        mesh=_SC_MESH,
        scratch_types=[
            pltpu.VMEM((16, 128), jnp.int32),
            pltpu.VMEM((16, 128), jnp.int32),
            pltpu.VMEM((BIN_ROWS, 128), jnp.int32),
            pltpu.VMEM((BIN_ROWS, 128), jnp.int32),
            pltpu.VMEM((BIN_ROWS, 128), jnp.int32),
            pltpu.VMEM((BIN_ROWS, 128), jnp.int32),
            pltpu.VMEM((16,), jnp.int32),
            pltpu.SemaphoreType.DMA,
        ],
        compiler_params=pltpu.CompilerParams(needs_layout_passes=False),
    )(src2d, dst2d)


def _cnt_body(dst_hbm, out_hbm, sh_cnt, dst_v, ones_v, stage_v):
    c = lax.axis_index("c")
    s = lax.axis_index("s")
    wid = s * NC + c
    izero = jnp.zeros((16,), jnp.int32)
    for g in range(8):
        ones_v[pl.ds(g * 16, 16)] = jnp.ones((16,), jnp.int32)
    for k in range(CNT_SUB // 16):
        stage_v[pl.ds(k * 16, 16)] = izero
    pltpu.sync_copy(stage_v, sh_cnt.at[pl.ds(s * CNT_SUB, CNT_SUB)])
    plsc.subcore_barrier()

    rows_per_tile = E_ROWS // (NC * NS)
    pltpu.sync_copy(dst_hbm.at[pl.ds(wid * rows_per_tile, rows_per_tile)],
                    dst_v)

    def row_body(r, _):
        pltpu.sync_copy(ones_v, sh_cnt.at[dst_v.at[r]], add=True)
        return 0
    lax.fori_loop(0, rows_per_tile, row_body, 0)
    plsc.subcore_barrier()
    pltpu.sync_copy(sh_cnt.at[pl.ds(s * CNT_SUB, CNT_SUB)], stage_v)
    pltpu.sync_copy(stage_v,
                    out_hbm.at[pl.ds(c * CNT_PAD + s * CNT_SUB, CNT_SUB)])


_SC_MESH = plsc.VectorSubcoreMesh(core_axis_name="c", subcore_axis_name="s")


@jax.jit
def _agg(table, bins, cnts):
    return pl.kernel(
        _agg_body,
        out_type=jax.ShapeDtypeStruct((ND_PAD, D), jnp.float32),
        mesh=_SC_MESH,
        scratch_types=[
            pltpu.VMEM_SHARED((RNG_ROWS + 16, D), jnp.float32),
            pltpu.VMEM((16, 128), jnp.int32),
            pltpu.VMEM((16, 128), jnp.int32),
            pltpu.VMEM((32,), jnp.int32),
            pltpu.VMEM((SEL_ROWS, 128), jnp.int32),
            pltpu.VMEM((128, D), jnp.float32),
            pltpu.VMEM((128, D), jnp.float32),
            pltpu.VMEM((128,), jnp.int32),
            pltpu.VMEM((128,), jnp.int32),
            pltpu.VMEM((128,), jnp.int32),
            pltpu.VMEM((128,), jnp.int32),
            pltpu.VMEM((ZB_ROWS, D), jnp.float32),
            pltpu.VMEM((ZB_ROWS, D), jnp.float32),
            pltpu.SemaphoreType.DMA,
        ],
        compiler_params=pltpu.CompilerParams(needs_layout_passes=False),
    )(table, bins, cnts)


@jax.jit
def _count(dst2d):
    return pl.kernel(
        _cnt_body,
        out_type=jax.ShapeDtypeStruct((NC * CNT_PAD,), jnp.int32),
        mesh=_SC_MESH,
        scratch_types=[
            pltpu.VMEM_SHARED((CNT_PAD,), jnp.int32),
            pltpu.VMEM((E_ROWS // (NC * NS), 128), jnp.int32),
            pltpu.VMEM((128,), jnp.int32),
            pltpu.VMEM((CNT_SUB,), jnp.int32),
        ],
    )(dst2d)


def _tc_body(cnt_ref, agg_ref, x_ref, wl_ref, bl_ref, wr_ref, o_ref,
             *, relu, nblk_agg):
    i = pl.program_id(0)
    valid = (i < nblk_agg).astype(jnp.float32)
    cnt = (cnt_ref[0, :] + cnt_ref[1, :]).astype(jnp.float32)
    inv = valid / jnp.maximum(cnt, 1.0)
    agg = agg_ref[...] * inv[:, None]
    acc = jnp.dot(agg, wl_ref[...], preferred_element_type=jnp.float32)
    acc = acc + jnp.dot(x_ref[...], wr_ref[...],
                        preferred_element_type=jnp.float32)
    acc = acc + bl_ref[...]
    if relu:
        acc = jnp.maximum(acc, 0.0)
    o_ref[...] = acc


def _tc_call(cnt, agg, x, Wl, bl, Wr, relu):
    n = x.shape[0]
    nblk_agg = ND_PAD // BM
    grid = pl.cdiv(n, BM)
    clamp = lambda i: jnp.minimum(i, nblk_agg - 1)
    return pl.pallas_call(
        functools.partial(_tc_body, relu=relu, nblk_agg=nblk_agg),
        grid=(grid,),
        in_specs=[
            pl.BlockSpec((2, BM), lambda i: (0, clamp(i))),
            pl.BlockSpec((BM, D), lambda i: (clamp(i), 0)),
            pl.BlockSpec((BM, D), lambda i: (i, 0)),
            pl.BlockSpec((D, D), lambda i: (0, 0)),
            pl.BlockSpec((1, D), lambda i: (0, 0)),
            pl.BlockSpec((D, D), lambda i: (0, 0)),
        ],
        out_specs=pl.BlockSpec((BM, D), lambda i: (i, 0)),
        out_shape=jax.ShapeDtypeStruct((n, D), jnp.float32),
    )(cnt, agg, x, Wl, bl, Wr)


def _pad_edges(edge_index):
    src = jnp.concatenate(
        [edge_index[0], jnp.zeros((E_PAD - E,), jnp.int32)]).reshape(E_ROWS, 128)
    dst = jnp.concatenate(
        [edge_index[1], jnp.full((E_PAD - E,), PAD_DST, jnp.int32)]
    ).reshape(E_ROWS, 128)
    return src, dst


def kernel(x_user, x_movie, edge_index_rates, edge_index_rev_rates,
           W1rl, b1rl, W1rr, W1vl, b1vl, W1vr,
           W2rl, b2rl, W2rr, W2vl, b2vl, W2vr):
    src_r, dst_r = _pad_edges(edge_index_rates)
    src_v, dst_v = _pad_edges(edge_index_rev_rates)
    cnt_r = _count(dst_r).reshape(NC, CNT_PAD)
    cnt_v = _count(dst_v).reshape(NC, CNT_PAD)

    bins_r, bcnt_r = _bin(src_r, dst_r)
    bins_v, bcnt_v = _bin(src_v, dst_v)

    agg1m = _agg(x_user, bins_r, bcnt_r)
    agg1u = _agg(x_movie, bins_v, bcnt_v)
    movie1 = _tc_call(cnt_r, agg1m, x_movie, W1rl, b1rl.reshape(1, D), W1rr,
                      relu=True)
    user1 = _tc_call(cnt_v, agg1u, x_user, W1vl, b1vl.reshape(1, D), W1vr,
                     relu=True)

    agg2m = _agg(user1, bins_r, bcnt_r)
    agg2u = _agg(movie1, bins_v, bcnt_v)
    movie2 = _tc_call(cnt_r, agg2m, movie1, W2rl, b2rl.reshape(1, D), W2rr,
                      relu=False)
    user2 = _tc_call(cnt_v, agg2u, user1, W2vl, b2vl.reshape(1, D), W2vr,
                     relu=False)
    return (user2, movie2)


# speedup vs baseline: 1.1310x; 1.0184x over previous
import functools
import jax
import jax.numpy as jnp
from jax import lax
from jax.experimental import pallas as pl
from jax.experimental.pallas import tpu as pltpu
from jax.experimental.pallas import tpu_sc as plsc

D = 128
N_USER = 100000
N_MOVIE = 50000
E = 500000

NC, NS = 2, 16
ND_PAD = 50176
NPASS = 4
RNG_ROWS = 6272
SUB_ROWS = RNG_ROWS // NS
ZB_ROWS = 56
TRASH = RNG_ROWS

E_PAD = 524288
E_ROWS = E_PAD // 128
CHUNK_ROWS = E_ROWS // NS
SR_ROWS = 128
BLK_ROWS = 16
NBLK = SR_ROWS // BLK_ROWS
SEL_ROWS = SR_ROWS + 8

BIN_ROWS = 128
PADV = (PAD_DST := 50176) and ((50176 << 16) - (1 << 32))

CNT_PAD = 50432
CNT_SUB = CNT_PAD // NS

BM = 2048


def _agg_body(table_hbm, bins_hbm, cnts_hbm, out_hbm,
              acc_sh, binst0_v, binst1_v, cnts_v, sel_v, rows0_v, rows1_v,
              isrc0_v, idst0_v, isrc1_v, idst1_v, wb0_v, wb1_v, sem):
    c = lax.axis_index("c")
    s = lax.axis_index("s")
    zvec = jnp.zeros((16,), jnp.float32)
    ipad = jnp.full((16,), TRASH << 16, jnp.int32)
    iota = lax.iota(jnp.int32, 16)
    rows_b = (rows0_v, rows1_v)
    isrc_b = (isrc0_v, isrc1_v)
    idst_b = (idst0_v, idst1_v)

    def unpack(j, b):
        for g in range(8):
            sl = pl.ds(g * 16, 16)
            packed = sel_v[j, sl]
            isrc_b[b][sl] = jnp.bitwise_and(packed, 0xFFFF)
            idst_b[b][sl] = lax.shift_right_logical(packed, 16)

    def start_gather(b):
        return pltpu.async_copy(table_hbm.at[isrc_b[b]], rows_b[b], sem)

    def wait_gather(b):
        pltpu.make_async_copy(table_hbm.at[isrc_b[b]], rows_b[b], sem).wait()

    wb_b = (wb0_v, wb1_v)

    def zfill(r, _):
        for g in range(8):
            wb0_v[r, pl.ds(g * 16, 16)] = zvec
        return 0
    lax.fori_loop(0, ZB_ROWS, zfill, 0)

    pltpu.sync_copy(cnts_hbm.at[pl.ds(s * 32, 32)], cnts_v)

    nzb = SUB_ROWS // ZB_ROWS
    for p in range(NPASS):
        rng = p * NC + c
        lo = rng * RNG_ROWS
        for t in range(nzb):
            pltpu.async_copy(
                wb0_v, acc_sh.at[pl.ds(s * SUB_ROWS + t * ZB_ROWS, ZB_ROWS)],
                sem)
        for t in range(nzb):
            pltpu.make_async_copy(
                wb0_v, acc_sh.at[pl.ds(s * SUB_ROWS, ZB_ROWS)], sem).wait()
        plsc.subcore_barrier()

        lo16 = rng * (RNG_ROWS << 16)
        for u in range(2):
            w = 2 * s + u
            cvec = cnts_v[pl.ds(u * 16, 16)]
            cnt = jnp.max(jnp.where(iota == p, cvec, 0))
            nblk = (cnt + 2047) // 2048
            bin_base = (w * 4 + p) * BIN_ROWS
            binst_b = (binst0_v, binst1_v)

            def bin_load(b, h):
                pltpu.async_copy(bins_hbm.at[pl.ds(bin_base + b * 16, 16)],
                                 binst_b[h], sem)

            def bin_wait(h):
                pltpu.make_async_copy(bins_hbm.at[pl.ds(bin_base, 16)],
                                      binst_b[h], sem).wait()

            @pl.when(nblk > 0)
            def _bprime():
                bin_load(0, 0)

            def blk2_body(bb, cur):
                for h in range(2):
                    b = bb * 2 + h

                    def scan_blk(cur_in):
                        bin_wait(h)

                        @pl.when(b + 1 < nblk)
                        def _pref():
                            bin_load(b + 1, 1 - h)

                        def vec_body(i, cur2):
                            r = i // 8
                            g = i % 8
                            pk = binst_b[h][r, pl.ds(g * 16, 16)]
                            d = lax.shift_right_logical(pk, 16)
                            ud = d - lo
                            m = (plsc.bitcast(ud, jnp.uint32)
                                 < jnp.uint32(RNG_ROWS))
                            cum = plsc.cumsum(m.astype(jnp.int32))
                            pos = cur2 + cum - 1
                            row = jnp.right_shift(pos, 7)
                            col = jnp.bitwise_and(pos, 127)
                            plsc.store_scatter(sel_v, [row, col], pk - lo16,
                                               mask=m)
                            return cur2 + plsc.all_reduce_population_count(m)
                        return lax.fori_loop(0, 128, vec_body, cur_in)
                    cur = lax.cond(b < nblk, scan_blk, lambda x: x, cur)
                return cur
            cursor = lax.fori_loop(0, (nblk + 1) // 2, blk2_body,
                                   jnp.zeros((16,), jnp.int32))

            n_sel = jnp.max(cursor)
            for k in range(8):
                pos = n_sel + k * 16 + iota
                row = jnp.right_shift(pos, 7)
                col = jnp.bitwise_and(pos, 127)
                plsc.store_scatter(sel_v, [row, col], ipad)

            nb = (n_sel + 127) // 128

            @pl.when(nb > 0)
            def _prime():
                unpack(0, 0)
                start_gather(0)

            def pair_body(jj, _):
                for b in range(2):
                    j = jj * 2 + b

                    @pl.when(j < nb)
                    def _step():
                        @pl.when(j + 1 < nb)
                        def _next():
                            unpack(j + 1, 1 - b)
                            start_gather(1 - b)
                        wait_gather(b)
                        pltpu.sync_copy(rows_b[b], acc_sh.at[idst_b[b]],
                                        add=True)
                return 0
            lax.fori_loop(0, (nb + 1) // 2, pair_body, 0)
        plsc.subcore_barrier()

        for t in range(nzb):
            b = t % 2
            off = s * SUB_ROWS + t * ZB_ROWS
            if t >= 2:
                pltpu.make_async_copy(
                    wb_b[b], out_hbm.at[pl.ds(lo, ZB_ROWS)], sem).wait()
            pltpu.sync_copy(acc_sh.at[pl.ds(off, ZB_ROWS)], wb_b[b])
            pltpu.async_copy(wb_b[b], out_hbm.at[pl.ds(lo + off, ZB_ROWS)],
                             sem)
        for b in range(2):
            pltpu.make_async_copy(
                wb_b[b], out_hbm.at[pl.ds(lo, ZB_ROWS)], sem).wait()
        plsc.subcore_barrier()
        if p + 1 < NPASS:
            lax.fori_loop(0, ZB_ROWS, zfill, 0)


def _bin_body(src_hbm, dst_hbm, bins_hbm, cnts_hbm,
              src_v, dst_v, sel0, sel1, sel2, sel3, cnts_v, sem):
    c = lax.axis_index("c")
    s = lax.axis_index("s")
    w = s * NC + c
    sels = (sel0, sel1, sel2, sel3)
    iota = lax.iota(jnp.int32, 16)
    padv = jnp.full((16,), PADV, jnp.int32)

    def pfill(r, _):
        for q in range(4):
            for g in range(8):
                sels[q][r, pl.ds(g * 16, 16)] = padv
        return 0
    lax.fori_loop(0, BIN_ROWS, pfill, 0)

    cursors = (jnp.zeros((16,), jnp.int32),) * 4
    for b in range(8):
        base = w * 128 + b * 16
        pltpu.async_copy(src_hbm.at[pl.ds(base, 16)], src_v, sem)
        pltpu.async_copy(dst_hbm.at[pl.ds(base, 16)], dst_v, sem)
        pltpu.make_async_copy(src_hbm.at[pl.ds(base, 16)], src_v, sem).wait()
        pltpu.make_async_copy(dst_hbm.at[pl.ds(base, 16)], dst_v, sem).wait()

        def vec_body(i, curs):
            r = i // 8
            g = i % 8
            sl = pl.ds(g * 16, 16)
            d = dst_v[r, sl]
            sr = src_v[r, sl]
            pid = ((d >= 2 * RNG_ROWS).astype(jnp.int32)
                   + (d >= 4 * RNG_ROWS).astype(jnp.int32)
                   + (d >= 6 * RNG_ROWS).astype(jnp.int32))
            pk = jnp.bitwise_or(
                sr, lax.shift_left(d, jnp.full((16,), 16, jnp.int32)))
            new = []
            for q in range(4):
                mq = pid == q
                cum = plsc.cumsum(mq.astype(jnp.int32))
                pos = curs[q] + cum - 1
                row = jnp.right_shift(pos, 7)
                col = jnp.bitwise_and(pos, 127)
                plsc.store_scatter(sels[q], [row, col], pk, mask=mq)
                new.append(curs[q] + plsc.all_reduce_population_count(mq))
            return tuple(new)
        cursors = lax.fori_loop(0, 128, vec_body, cursors)

    cv = jnp.zeros((16,), jnp.int32)
    for q in range(4):
        cv = jnp.where(iota == q, cursors[q], cv)
    cnts_v[...] = cv
    pltpu.sync_copy(cnts_v, cnts_hbm.at[pl.ds(w * 16, 16)])
    for q in range(4):
        pltpu.sync_copy(sels[q],
                        bins_hbm.at[pl.ds((w * 4 + q) * BIN_ROWS, BIN_ROWS)])


@jax.jit
def _bin(src2d, dst2d):
    return pl.kernel(
        _bin_body,
        out_type=(jax.ShapeDtypeStruct((NC * NS * 4 * BIN_ROWS, 128),
                                       jnp.int32),
                  jax.ShapeDtypeStruct((NC * NS * 16,), jnp.int32)),
        mesh=_SC_MESH,
        scratch_types=[
            pltpu.VMEM((16, 128), jnp.int32),
            pltpu.VMEM((16, 128), jnp.int32),
            pltpu.VMEM((BIN_ROWS, 128), jnp.int32),
            pltpu.VMEM((BIN_ROWS, 128), jnp.int32),
            pltpu.VMEM((BIN_ROWS, 128), jnp.int32),
            pltpu.VMEM((BIN_ROWS, 128), jnp.int32),
            pltpu.VMEM((16,), jnp.int32),
            pltpu.SemaphoreType.DMA,
        ],
        compiler_params=pltpu.CompilerParams(needs_layout_passes=False),
    )(src2d, dst2d)


def _cnt_body(dst_hbm, out_hbm, sh_cnt, dst_v, ones_v, stage_v):
    c = lax.axis_index("c")
    s = lax.axis_index("s")
    wid = s * NC + c
    izero = jnp.zeros((16,), jnp.int32)
    for g in range(8):
        ones_v[pl.ds(g * 16, 16)] = jnp.ones((16,), jnp.int32)
    for k in range(CNT_SUB // 16):
        stage_v[pl.ds(k * 16, 16)] = izero
    pltpu.sync_copy(stage_v, sh_cnt.at[pl.ds(s * CNT_SUB, CNT_SUB)])
    plsc.subcore_barrier()

    rows_per_tile = E_ROWS // (NC * NS)
    pltpu.sync_copy(dst_hbm.at[pl.ds(wid * rows_per_tile, rows_per_tile)],
                    dst_v)

    def row_body(r, _):
        pltpu.sync_copy(ones_v, sh_cnt.at[dst_v.at[r]], add=True)
        return 0
    lax.fori_loop(0, rows_per_tile, row_body, 0)
    plsc.subcore_barrier()
    pltpu.sync_copy(sh_cnt.at[pl.ds(s * CNT_SUB, CNT_SUB)], stage_v)
    pltpu.sync_copy(stage_v,
                    out_hbm.at[pl.ds(c * CNT_PAD + s * CNT_SUB, CNT_SUB)])


_SC_MESH = plsc.VectorSubcoreMesh(core_axis_name="c", subcore_axis_name="s")


@jax.jit
def _agg(table, bins, cnts):
    return pl.kernel(
        _agg_body,
        out_type=jax.ShapeDtypeStruct((ND_PAD, D), jnp.float32),
        mesh=_SC_MESH,
        scratch_types=[
            pltpu.VMEM_SHARED((RNG_ROWS + 16, D), jnp.float32),
            pltpu.VMEM((16, 128), jnp.int32),
            pltpu.VMEM((16, 128), jnp.int32),
            pltpu.VMEM((32,), jnp.int32),
            pltpu.VMEM((SEL_ROWS, 128), jnp.int32),
            pltpu.VMEM((128, D), jnp.float32),
            pltpu.VMEM((128, D), jnp.float32),
            pltpu.VMEM((128,), jnp.int32),
            pltpu.VMEM((128,), jnp.int32),
            pltpu.VMEM((128,), jnp.int32),
            pltpu.VMEM((128,), jnp.int32),
            pltpu.VMEM((ZB_ROWS, D), jnp.float32),
            pltpu.VMEM((ZB_ROWS, D), jnp.float32),
            pltpu.SemaphoreType.DMA,
        ],
        compiler_params=pltpu.CompilerParams(needs_layout_passes=False),
    )(table, bins, cnts)


@jax.jit
def _count(dst2d):
    return pl.kernel(
        _cnt_body,
        out_type=jax.ShapeDtypeStruct((NC * CNT_PAD,), jnp.int32),
        mesh=_SC_MESH,
        scratch_types=[
            pltpu.VMEM_SHARED((CNT_PAD,), jnp.int32),
            pltpu.VMEM((E_ROWS // (NC * NS), 128), jnp.int32),
            pltpu.VMEM((128,), jnp.int32),
            pltpu.VMEM((CNT_SUB,), jnp.int32),
        ],
    )(dst2d)


def _tc_body(cnt_ref, agg_ref, x_ref, wl_ref, bl_ref, wr_ref, o_ref,
             *, relu, nblk_agg):
    i = pl.program_id(0)
    valid = (i < nblk_agg).astype(jnp.float32)
    cnt = (cnt_ref[0, :] + cnt_ref[1, :]).astype(jnp.float32)
    inv = valid / jnp.maximum(cnt, 1.0)
    agg = agg_ref[...] * inv[:, None]
    acc = jnp.dot(agg, wl_ref[...], preferred_element_type=jnp.float32)
    acc = acc + jnp.dot(x_ref[...], wr_ref[...],
                        preferred_element_type=jnp.float32)
    acc = acc + bl_ref[...]
    if relu:
        acc = jnp.maximum(acc, 0.0)
    o_ref[...] = acc


def _tc_call(cnt, agg, x, Wl, bl, Wr, relu):
    n = x.shape[0]
    nblk_agg = ND_PAD // BM
    grid = pl.cdiv(n, BM)
    clamp = lambda i: jnp.minimum(i, nblk_agg - 1)
    return pl.pallas_call(
        functools.partial(_tc_body, relu=relu, nblk_agg=nblk_agg),
        grid=(grid,),
        in_specs=[
            pl.BlockSpec((2, BM), lambda i: (0, clamp(i))),
            pl.BlockSpec((BM, D), lambda i: (clamp(i), 0)),
            pl.BlockSpec((BM, D), lambda i: (i, 0)),
            pl.BlockSpec((D, D), lambda i: (0, 0)),
            pl.BlockSpec((1, D), lambda i: (0, 0)),
            pl.BlockSpec((D, D), lambda i: (0, 0)),
        ],
        out_specs=pl.BlockSpec((BM, D), lambda i: (i, 0)),
        out_shape=jax.ShapeDtypeStruct((n, D), jnp.float32),
    )(cnt, agg, x, Wl, bl, Wr)


def _pad_edges(edge_index):
    src = jnp.concatenate(
        [edge_index[0], jnp.zeros((E_PAD - E,), jnp.int32)]).reshape(E_ROWS, 128)
    dst = jnp.concatenate(
        [edge_index[1], jnp.full((E_PAD - E,), PAD_DST, jnp.int32)]
    ).reshape(E_ROWS, 128)
    return src, dst


def kernel(x_user, x_movie, edge_index_rates, edge_index_rev_rates,
           W1rl, b1rl, W1rr, W1vl, b1vl, W1vr,
           W2rl, b2rl, W2rr, W2vl, b2vl, W2vr):
    src_r, dst_r = _pad_edges(edge_index_rates)
    src_v, dst_v = _pad_edges(edge_index_rev_rates)
    cnt_r = _count(dst_r).reshape(NC, CNT_PAD)
    cnt_v = _count(dst_v).reshape(NC, CNT_PAD)

    bins_r, bcnt_r = _bin(src_r, dst_r)
    bins_v, bcnt_v = _bin(src_v, dst_v)

    agg1m = _agg(x_user, bins_r, bcnt_r)
    agg1u = _agg(x_movie, bins_v, bcnt_v)
    movie1 = _tc_call(cnt_r, agg1m, x_movie, W1rl, b1rl.reshape(1, D), W1rr,
                      relu=True)
    user1 = _tc_call(cnt_v, agg1u, x_user, W1vl, b1vl.reshape(1, D), W1vr,
                     relu=True)

    agg2m = _agg(user1, bins_r, bcnt_r)
    agg2u = _agg(movie1, bins_v, bcnt_v)
    movie2 = _tc_call(cnt_r, agg2m, movie1, W2rl, b2rl.reshape(1, D), W2rr,
                      relu=False)
    user2 = _tc_call(cnt_v, agg2u, user1, W2vl, b2vl.reshape(1, D), W2vr,
                     relu=False)
    return (user2, movie2)
